# Initial kernel scaffold; baseline (speedup 1.0000x reference)
import jax
import jax.numpy as jnp
from jax.experimental import pallas as pl


def kernel(z, a):
    raise NotImplementedError("write your pallas kernel here")


# speedup vs baseline: 6.0374x; 6.0374x over previous
import functools

import jax
import jax.numpy as jnp
from jax.experimental import pallas as pl

_ATTR_INDEX = 0
_BN = 4096


def _body(z_ref, a_ref, o_ref):
    zc = z_ref[:, _ATTR_INDEX][:, None]
    k = a_ref.shape[1]
    idx = jnp.clip((zc * k).astype(jnp.int32), 0, k - 1)
    lanes = jax.lax.broadcasted_iota(jnp.int32, a_ref.shape, 1)
    picked = jnp.where(lanes == idx, a_ref[...], 0.0)
    o_ref[...] = jnp.sum(picked, axis=1) * 0.999


def kernel(z, a):
    b, d = z.shape
    _, k = a.shape
    grid = (b // _BN,)
    return pl.pallas_call(
        _body,
        grid=grid,
        in_specs=[
            pl.BlockSpec((_BN, d), lambda i: (i, 0)),
            pl.BlockSpec((_BN, k), lambda i: (i, 0)),
        ],
        out_specs=pl.BlockSpec((_BN,), lambda i: (i,)),
        out_shape=jax.ShapeDtypeStruct((b,), jnp.float32),
    )(z, a)


# speedup vs baseline: 8.1758x; 1.3542x over previous
import functools

import jax
import jax.numpy as jnp
from jax import lax
from jax.experimental import pallas as pl
from jax.experimental.pallas import tpu as pltpu
from jax.experimental.pallas import tpu_sc as plsc

_NC = 2
_NS = 16
_NW = _NC * _NS
_L = 16
_CH = 2048
_GW = 128
_NG = _CH // _GW


def _sc_body(bpw, d, k, z_hbm, a_hbm, out_hbm, zbuf, idxbuf, gbuf, sem):
    wid = lax.axis_index("s") * _NC + lax.axis_index("c")
    base = wid * bpw
    kf = jnp.float32(k)
    kmax = jnp.int32(k - 1)

    def chunk(ci, _):
        cbase = base + ci * _CH
        pltpu.sync_copy(z_hbm.at[pl.ds(cbase * d, _CH * d)], zbuf)

        def idx_step(g, _):
            for l in range(_GW // _L):
                r0 = g * _GW + l * _L
                rows = lax.iota(jnp.int32, _L) + r0
                zc = plsc.load_gather(zbuf, [rows * d])
                idx = jnp.clip((zc * kf).astype(jnp.int32), 0, kmax)
                flat = (cbase + r0 + lax.iota(jnp.int32, _L)) * k + idx
                idxbuf[g, pl.ds(l * _L, _L)] = flat
            return 0

        lax.fori_loop(0, _NG, idx_step, 0)

        copies = [
            pltpu.make_async_copy(
                a_hbm.at[idxbuf.at[g]], gbuf.at[pl.ds(g * _GW, _GW)], sem
            )
            for g in range(_NG)
        ]
        for c in copies:
            c.start()
        for c in copies:
            c.wait()

        def scale_step(v, _):
            gbuf[pl.ds(v * _L, _L)] = gbuf[pl.ds(v * _L, _L)] * 0.999
            return 0

        lax.fori_loop(0, _CH // _L, scale_step, 0)
        pltpu.sync_copy(gbuf, out_hbm.at[pl.ds(cbase, _CH)])
        return 0

    lax.fori_loop(0, bpw // _CH, chunk, 0)


def kernel(z, a):
    b, d = z.shape
    _, k = a.shape
    bpw = b // _NW
    assert b % (_NW * _CH) == 0
    z_flat = z.reshape(-1)
    a_flat = a.reshape(-1)
    mesh = plsc.VectorSubcoreMesh(
        core_axis_name="c", subcore_axis_name="s", num_cores=_NC, num_subcores=_NS
    )
    fn = pl.kernel(
        functools.partial(_sc_body, bpw, d, k),
        out_type=jax.ShapeDtypeStruct((b,), jnp.float32),
        mesh=mesh,
        compiler_params=pltpu.CompilerParams(needs_layout_passes=False),
        scratch_types=[
            pltpu.VMEM((_CH * d,), jnp.float32),
            pltpu.VMEM((_NG, _GW), jnp.int32),
            pltpu.VMEM((_CH,), jnp.float32),
            pltpu.SemaphoreType.DMA,
        ],
    )
    return fn(z_flat, a_flat)


# speedup vs baseline: 8.7102x; 1.0654x over previous
import functools

import jax
import jax.numpy as jnp
from jax import lax
from jax.experimental import pallas as pl
from jax.experimental.pallas import tpu as pltpu
from jax.experimental.pallas import tpu_sc as plsc

_NC = 2
_NS = 16
_NW = _NC * _NS
_L = 16
_CH = 2048
_GW = 128
_NG = _CH // _GW


def _sc_body(bpw, d, k, z_hbm, a_hbm, out_hbm, zbuf, idxbuf, gbuf, zsem, gsem, osem):
    wid = lax.axis_index("s") * _NC + lax.axis_index("c")
    base = wid * bpw
    kf = jnp.float32(k)
    kmax = jnp.int32(k - 1)
    nch = bpw // _CH

    def zcopy(ci, b):
        return pltpu.make_async_copy(
            z_hbm.at[pl.ds((base + ci * _CH) * d, _CH * d)],
            zbuf.at[pl.ds(b * _CH * d, _CH * d)],
            zsem,
        )

    def gdrain(b):
        return pltpu.make_async_copy(
            a_hbm.at[idxbuf.at[b, 0]], gbuf.at[b, pl.ds(0, _GW)], gsem
        )

    def odrain(b):
        return pltpu.make_async_copy(
            gbuf.at[b], out_hbm.at[pl.ds(base, _CH)], osem
        )

    zcopy(0, 0).start()

    def chunk_work(ci, b):
        cbase = base + ci * _CH
        zcopy(ci, b).wait()

        @pl.when(ci + 1 < nch)
        def _():
            zcopy(ci + 1, 1 - b).start()

        @pl.when(ci >= 2)
        def _():
            odrain(b).wait()

        def idx_step(g, _):
            for l in range(_GW // _L):
                r0 = g * _GW + l * _L
                rows = lax.iota(jnp.int32, _L) + r0
                zc = plsc.load_gather(zbuf, [rows * d + b * _CH * d])
                idx = jnp.clip((zc * kf).astype(jnp.int32), 0, kmax)
                flat = (cbase + r0 + lax.iota(jnp.int32, _L)) * k + idx
                idxbuf[b, g, pl.ds(l * _L, _L)] = flat
            pltpu.make_async_copy(
                a_hbm.at[idxbuf.at[b, g]], gbuf.at[b, pl.ds(g * _GW, _GW)], gsem
            ).start()
            return 0

        lax.fori_loop(0, _NG, idx_step, 0)

        def drain_step(g, _):
            gdrain(b).wait()
            return 0

        lax.fori_loop(0, _NG, drain_step, 0)

        def scale_step(g, _):
            for l in range(_GW // _L):
                o = g * _GW + l * _L
                gbuf[b, pl.ds(o, _L)] = gbuf[b, pl.ds(o, _L)] * 0.999
            return 0

        lax.fori_loop(0, _NG, scale_step, 0)
        pltpu.make_async_copy(
            gbuf.at[b], out_hbm.at[pl.ds(cbase, _CH)], osem
        ).start()

    def loop_body(i, _):
        chunk_work(2 * i, 0)
        chunk_work(2 * i + 1, 1)
        return 0

    lax.fori_loop(0, nch // 2, loop_body, 0)
    odrain(0).wait()
    odrain(1).wait()


def kernel(z, a):
    b, d = z.shape
    _, k = a.shape
    bpw = b // _NW
    assert b % (_NW * _CH * 2) == 0
    z_flat = z.reshape(-1)
    a_flat = a.reshape(-1)
    mesh = plsc.VectorSubcoreMesh(
        core_axis_name="c", subcore_axis_name="s", num_cores=_NC, num_subcores=_NS
    )
    fn = pl.kernel(
        functools.partial(_sc_body, bpw, d, k),
        out_type=jax.ShapeDtypeStruct((b,), jnp.float32),
        mesh=mesh,
        compiler_params=pltpu.CompilerParams(needs_layout_passes=False),
        scratch_types=[
            pltpu.VMEM((2 * _CH * d,), jnp.float32),
            pltpu.VMEM((2, _NG, _GW), jnp.int32),
            pltpu.VMEM((2, _CH), jnp.float32),
            pltpu.SemaphoreType.DMA,
            pltpu.SemaphoreType.DMA,
            pltpu.SemaphoreType.DMA,
        ],
    )
    return fn(z_flat, a_flat)
